# initial kernel scaffold (unmeasured)
import jax
import jax.numpy as jnp
from jax import lax
from jax.experimental import pallas as pl
from jax.experimental.pallas import tpu as pltpu

N_DEV = 4


def kernel(x, w_mat):
    m, k_per = x.shape
    k_per2, n = w_mat.shape
    assert k_per == k_per2
    m_chunk = m // N_DEV

    def body(x_ref, w_ref, out_ref, send_buf, rs_bufs, send_sem,
             rs_sems, ag_sems):
        my_pos = lax.axis_index("i")
        left = (my_pos - 1) % N_DEV
        right = (my_pos + 1) % N_DEV

        def chunk_partial(c):
            xs = x_ref[pl.ds(c * m_chunk, m_chunk), :]
            return jnp.dot(xs, w_ref[:, :], preferred_element_type=jnp.float32)

        barrier_sem = pltpu.get_barrier_semaphore()
        pl.semaphore_signal(barrier_sem, inc=1, device_id=(left,),
                            device_id_type=pl.DeviceIdType.MESH)
        pl.semaphore_signal(barrier_sem, inc=1, device_id=(right,),
                            device_id_type=pl.DeviceIdType.MESH)
        pl.semaphore_wait(barrier_sem, 2)

        send_buf[:, :] = chunk_partial(my_pos)
        for h in range(N_DEV - 1):
            rdma = pltpu.make_async_remote_copy(
                src_ref=send_buf,
                dst_ref=rs_bufs.at[h],
                send_sem=send_sem,
                recv_sem=rs_sems.at[h],
                device_id=(right,),
                device_id_type=pl.DeviceIdType.MESH,
            )
            rdma.start()
            rdma.wait()
            c = (my_pos - h - 1) % N_DEV
            acc = chunk_partial(c) + rs_bufs[h, :, :]
            if h < N_DEV - 2:
                send_buf[:, :] = acc
            else:
                y = acc
                silu = y * (1.0 / (1.0 + jnp.exp(-y)))
                out_ref[pl.ds(c * m_chunk, m_chunk), :] = silu

        for g in range(N_DEV - 1):
            send_c = (my_pos - g + 1) % N_DEV
            recv_c = (my_pos - g) % N_DEV
            rdma = pltpu.make_async_remote_copy(
                src_ref=out_ref.at[pl.ds(send_c * m_chunk, m_chunk), :],
                dst_ref=out_ref.at[pl.ds(send_c * m_chunk, m_chunk), :],
                send_sem=send_sem,
                recv_sem=ag_sems.at[g],
                device_id=(right,),
                device_id_type=pl.DeviceIdType.MESH,
            )
            rdma.start()
            rdma.wait()
            del recv_c

    return pl.pallas_call(
        body,
        out_shape=jax.ShapeDtypeStruct((m, n), jnp.float32),
        in_specs=[
            pl.BlockSpec(memory_space=pltpu.VMEM),
            pl.BlockSpec(memory_space=pltpu.VMEM),
        ],
        out_specs=pl.BlockSpec(memory_space=pltpu.VMEM),
        scratch_shapes=[
            pltpu.VMEM((m_chunk, n), jnp.float32),
            pltpu.VMEM((N_DEV - 1, m_chunk, n), jnp.float32),
            pltpu.SemaphoreType.DMA,
            pltpu.SemaphoreType.DMA((N_DEV - 1,)),
            pltpu.SemaphoreType.DMA((N_DEV - 1,)),
        ],
        compiler_params=pltpu.CompilerParams(collective_id=0),
    )(x, w_mat)


# baseline (device time: 310445 ns/iter reference)
import jax
import jax.numpy as jnp
from jax import lax
from jax.experimental import pallas as pl
from jax.experimental.pallas import tpu as pltpu

N_DEV = 4


def kernel(x, w_mat):
    m, k_per = x.shape
    k_per2, n = w_mat.shape
    assert k_per == k_per2
    m_chunk = m // N_DEV

    def body(x_ref, w_ref, out_ref, send_buf, rs_bufs, send_sem,
             rs_sems, ag_sems):
        my_pos = lax.axis_index("i")
        left = (my_pos - 1) % N_DEV
        right = (my_pos + 1) % N_DEV

        def chunk_partial(c):
            xs = x_ref[pl.ds(c * m_chunk, m_chunk), :]
            return jnp.dot(xs, w_ref[:, :], preferred_element_type=jnp.float32)

        barrier_sem = pltpu.get_barrier_semaphore()
        pl.semaphore_signal(barrier_sem, inc=1, device_id=(left,),
                            device_id_type=pl.DeviceIdType.MESH)
        pl.semaphore_signal(barrier_sem, inc=1, device_id=(right,),
                            device_id_type=pl.DeviceIdType.MESH)
        pl.semaphore_wait(barrier_sem, 2)

        send_buf[:, :] = chunk_partial(my_pos)
        for h in range(N_DEV - 1):
            rdma = pltpu.make_async_remote_copy(
                src_ref=send_buf,
                dst_ref=rs_bufs.at[h],
                send_sem=send_sem,
                recv_sem=rs_sems.at[h],
                device_id=(right,),
                device_id_type=pl.DeviceIdType.MESH,
            )
            rdma.start()
            rdma.wait()
            c = (my_pos - h - 1) % N_DEV
            acc = chunk_partial(c) + rs_bufs[h, :, :]
            if h < N_DEV - 2:
                send_buf[:, :] = acc
            else:
                y = acc
                silu = y * (1.0 / (1.0 + jnp.exp(-y)))
                out_ref[pl.ds(c * m_chunk, m_chunk), :] = silu

        for g in range(N_DEV - 1):
            send_c = (my_pos - g + 1) % N_DEV
            recv_c = (my_pos - g) % N_DEV
            rdma = pltpu.make_async_remote_copy(
                src_ref=out_ref.at[pl.ds(send_c * m_chunk, m_chunk), :],
                dst_ref=out_ref.at[pl.ds(send_c * m_chunk, m_chunk), :],
                send_sem=send_sem,
                recv_sem=ag_sems.at[g],
                device_id=(right,),
                device_id_type=pl.DeviceIdType.MESH,
            )
            rdma.start()
            rdma.wait()
            del recv_c

    return pl.pallas_call(
        body,
        out_shape=jax.ShapeDtypeStruct((m, n), jnp.float32),
        in_specs=[
            pl.BlockSpec(memory_space=pltpu.VMEM),
            pl.BlockSpec(memory_space=pltpu.VMEM),
        ],
        out_specs=pl.BlockSpec(memory_space=pltpu.VMEM),
        scratch_shapes=[
            pltpu.VMEM((m_chunk, n), jnp.float32),
            pltpu.VMEM((N_DEV - 1, m_chunk, n), jnp.float32),
            pltpu.SemaphoreType.DMA,
            pltpu.SemaphoreType.DMA((N_DEV - 1,)),
            pltpu.SemaphoreType.DMA((N_DEV - 1,)),
        ],
        compiler_params=pltpu.CompilerParams(
            collective_id=0,
            vmem_limit_bytes=100 * 1024 * 1024,
        ),
    )(x, w_mat)


# device time: 176031 ns/iter; 1.7636x vs baseline; 1.7636x over previous
import jax
import jax.numpy as jnp
from jax import lax
from jax.experimental import pallas as pl
from jax.experimental.pallas import tpu as pltpu

N_DEV = 4


def kernel(x, w_mat):
    m, k_per = x.shape
    k_per2, n = w_mat.shape
    assert k_per == k_per2
    m_chunk = m // N_DEV
    m_half = m_chunk // 2

    def body(x_ref, w_ref, out_ref, send_r, send_l, rs_r, rs_l,
             send_sem_r, send_sem_l, rs_sems_r, rs_sems_l,
             ag_sems_r, ag_sems_l):
        my_pos = lax.axis_index("i")
        left = (my_pos - 1) % N_DEV
        right = (my_pos + 1) % N_DEV

        def partial_half(c, half):
            xs = x_ref[pl.ds(c * m_chunk + half * m_half, m_half), :]
            return jnp.dot(xs, w_ref[:, :], preferred_element_type=jnp.float32)

        barrier_sem = pltpu.get_barrier_semaphore()
        pl.semaphore_signal(barrier_sem, inc=1, device_id=(left,),
                            device_id_type=pl.DeviceIdType.MESH)
        pl.semaphore_signal(barrier_sem, inc=1, device_id=(right,),
                            device_id_type=pl.DeviceIdType.MESH)
        pl.semaphore_wait(barrier_sem, 2)

        send_r[:, :] = partial_half(my_pos, 0)
        send_l[:, :] = partial_half(my_pos, 1)
        for h in range(N_DEV - 1):
            rdma_r = pltpu.make_async_remote_copy(
                src_ref=send_r, dst_ref=rs_r.at[h],
                send_sem=send_sem_r, recv_sem=rs_sems_r.at[h],
                device_id=(right,), device_id_type=pl.DeviceIdType.MESH,
            )
            rdma_l = pltpu.make_async_remote_copy(
                src_ref=send_l, dst_ref=rs_l.at[h],
                send_sem=send_sem_l, recv_sem=rs_sems_l.at[h],
                device_id=(left,), device_id_type=pl.DeviceIdType.MESH,
            )
            rdma_r.start()
            rdma_l.start()
            cr = (my_pos - h - 1) % N_DEV
            cl = (my_pos + h + 1) % N_DEV
            pr = partial_half(cr, 0)
            pl_ = partial_half(cl, 1)
            rdma_r.wait()
            rdma_l.wait()
            acc_r = pr + rs_r[h, :, :]
            acc_l = pl_ + rs_l[h, :, :]
            if h < N_DEV - 2:
                send_r[:, :] = acc_r
                send_l[:, :] = acc_l
            else:
                out_ref[pl.ds(cr * m_chunk, m_half), :] = (
                    acc_r * (1.0 / (1.0 + jnp.exp(-acc_r))))
                out_ref[pl.ds(cl * m_chunk + m_half, m_half), :] = (
                    acc_l * (1.0 / (1.0 + jnp.exp(-acc_l))))

        for g in range(N_DEV - 1):
            sr = (my_pos - g + 1) % N_DEV
            sl = (my_pos + g - 1) % N_DEV
            rdma_r = pltpu.make_async_remote_copy(
                src_ref=out_ref.at[pl.ds(sr * m_chunk, m_half), :],
                dst_ref=out_ref.at[pl.ds(sr * m_chunk, m_half), :],
                send_sem=send_sem_r, recv_sem=ag_sems_r.at[g],
                device_id=(right,), device_id_type=pl.DeviceIdType.MESH,
            )
            rdma_l = pltpu.make_async_remote_copy(
                src_ref=out_ref.at[pl.ds(sl * m_chunk + m_half, m_half), :],
                dst_ref=out_ref.at[pl.ds(sl * m_chunk + m_half, m_half), :],
                send_sem=send_sem_l, recv_sem=ag_sems_l.at[g],
                device_id=(left,), device_id_type=pl.DeviceIdType.MESH,
            )
            rdma_r.start()
            rdma_l.start()
            rdma_r.wait()
            rdma_l.wait()

    return pl.pallas_call(
        body,
        out_shape=jax.ShapeDtypeStruct((m, n), jnp.float32),
        in_specs=[
            pl.BlockSpec(memory_space=pltpu.VMEM),
            pl.BlockSpec(memory_space=pltpu.VMEM),
        ],
        out_specs=pl.BlockSpec(memory_space=pltpu.VMEM),
        scratch_shapes=[
            pltpu.VMEM((m_half, n), jnp.float32),
            pltpu.VMEM((m_half, n), jnp.float32),
            pltpu.VMEM((N_DEV - 1, m_half, n), jnp.float32),
            pltpu.VMEM((N_DEV - 1, m_half, n), jnp.float32),
            pltpu.SemaphoreType.DMA,
            pltpu.SemaphoreType.DMA,
            pltpu.SemaphoreType.DMA((N_DEV - 1,)),
            pltpu.SemaphoreType.DMA((N_DEV - 1,)),
            pltpu.SemaphoreType.DMA((N_DEV - 1,)),
            pltpu.SemaphoreType.DMA((N_DEV - 1,)),
        ],
        compiler_params=pltpu.CompilerParams(
            collective_id=0,
            vmem_limit_bytes=100 * 1024 * 1024,
        ),
    )(x, w_mat)


# device time: 166102 ns/iter; 1.8690x vs baseline; 1.0598x over previous
import jax
import jax.numpy as jnp
from jax import lax
from jax.experimental import pallas as pl
from jax.experimental.pallas import tpu as pltpu

N_DEV = 4
SEG = 2


def kernel(x, w_mat):
    m, k_per = x.shape
    k_per2, n = w_mat.shape
    assert k_per == k_per2
    m_chunk = m // N_DEV
    m_half = m_chunk // 2
    m_seg = m_half // SEG
    n_hops = N_DEV - 1

    def body(x_ref, w_ref, out_ref, acc_r, acc_l, rcv_r, rcv_l,
             rs_ssem_r, rs_ssem_l, rs_rsem_r, rs_rsem_l,
             ag_ssem_r, ag_ssem_l, ag_rsem_r, ag_rsem_l):
        my_pos = lax.axis_index("i")
        left = (my_pos - 1) % N_DEV
        right = (my_pos + 1) % N_DEV

        def partial_rows(row0, nrows):
            xs = x_ref[pl.ds(row0, nrows), :]
            return jnp.dot(xs, w_ref[:, :], preferred_element_type=jnp.float32)

        def silu(y):
            return y * (1.0 / (1.0 + jnp.exp(-y)))

        def slab_row0(c, dirn):
            return c * m_chunk + dirn * m_half

        def rs_desc(dirn, h, s):
            acc, rcv = (acc_r, rcv_r) if dirn == 0 else (acc_l, rcv_l)
            ssem = rs_ssem_r if dirn == 0 else rs_ssem_l
            rsem = rs_rsem_r if dirn == 0 else rs_rsem_l
            tgt = right if dirn == 0 else left
            return pltpu.make_async_remote_copy(
                src_ref=acc.at[h, pl.ds(s * m_seg, m_seg), :],
                dst_ref=rcv.at[h, pl.ds(s * m_seg, m_seg), :],
                send_sem=ssem.at[h, s],
                recv_sem=rsem.at[h, s],
                device_id=(tgt,),
                device_id_type=pl.DeviceIdType.MESH,
            )

        def ag_desc(dirn, g, s):
            if dirn == 0:
                sc = (my_pos - g + 1) % N_DEV
                tgt = right
                ssem, rsem = ag_ssem_r, ag_rsem_r
            else:
                sc = (my_pos + g - 1) % N_DEV
                tgt = left
                ssem, rsem = ag_ssem_l, ag_rsem_l
            row0 = slab_row0(sc, dirn) + s * m_seg
            sl = out_ref.at[pl.ds(row0, m_seg), :]
            return pltpu.make_async_remote_copy(
                src_ref=sl, dst_ref=sl,
                send_sem=ssem.at[g, s],
                recv_sem=rsem.at[g, s],
                device_id=(tgt,),
                device_id_type=pl.DeviceIdType.MESH,
            )

        acc_r[0, :, :] = partial_rows(slab_row0(my_pos, 0), m_half)
        acc_l[0, :, :] = partial_rows(slab_row0(my_pos, 1), m_half)

        barrier_sem = pltpu.get_barrier_semaphore()
        pl.semaphore_signal(barrier_sem, inc=1, device_id=(left,),
                            device_id_type=pl.DeviceIdType.MESH)
        pl.semaphore_signal(barrier_sem, inc=1, device_id=(right,),
                            device_id_type=pl.DeviceIdType.MESH)
        pl.semaphore_wait(barrier_sem, 2)

        for s in range(SEG):
            rs_desc(0, 0, s).start()
            rs_desc(1, 0, s).start()
        for h in range(n_hops):
            cr = (my_pos - h - 1) % N_DEV
            cl = (my_pos + h + 1) % N_DEV
            for s in range(SEG):
                for dirn, c in ((0, cr), (1, cl)):
                    row0 = slab_row0(c, dirn) + s * m_seg
                    p = partial_rows(row0, m_seg)
                    rs_desc(dirn, h, s).wait_recv()
                    rcv = rcv_r if dirn == 0 else rcv_l
                    val = p + rcv[h, pl.ds(s * m_seg, m_seg), :]
                    if h < n_hops - 1:
                        acc = acc_r if dirn == 0 else acc_l
                        acc[h + 1, pl.ds(s * m_seg, m_seg), :] = val
                        rs_desc(dirn, h + 1, s).start()
                    else:
                        out_ref[pl.ds(row0, m_seg), :] = silu(val)

        for s in range(SEG):
            ag_desc(0, 0, s).start()
            ag_desc(1, 0, s).start()
        for g in range(n_hops):
            for s in range(SEG):
                for dirn in (0, 1):
                    ag_desc(dirn, g, s).wait_recv()
                    if g < n_hops - 1:
                        ag_desc(dirn, g + 1, s).start()

        for h in range(n_hops):
            for s in range(SEG):
                for dirn in (0, 1):
                    rs_desc(dirn, h, s).wait_send()
                    ag_desc(dirn, h, s).wait_send()

    return pl.pallas_call(
        body,
        out_shape=jax.ShapeDtypeStruct((m, n), jnp.float32),
        in_specs=[
            pl.BlockSpec(memory_space=pltpu.VMEM),
            pl.BlockSpec(memory_space=pltpu.VMEM),
        ],
        out_specs=pl.BlockSpec(memory_space=pltpu.VMEM),
        scratch_shapes=[
            pltpu.VMEM((n_hops, m_half, n), jnp.float32),
            pltpu.VMEM((n_hops, m_half, n), jnp.float32),
            pltpu.VMEM((n_hops, m_half, n), jnp.float32),
            pltpu.VMEM((n_hops, m_half, n), jnp.float32),
            pltpu.SemaphoreType.DMA((n_hops, SEG)),
            pltpu.SemaphoreType.DMA((n_hops, SEG)),
            pltpu.SemaphoreType.DMA((n_hops, SEG)),
            pltpu.SemaphoreType.DMA((n_hops, SEG)),
            pltpu.SemaphoreType.DMA((n_hops, SEG)),
            pltpu.SemaphoreType.DMA((n_hops, SEG)),
            pltpu.SemaphoreType.DMA((n_hops, SEG)),
            pltpu.SemaphoreType.DMA((n_hops, SEG)),
        ],
        compiler_params=pltpu.CompilerParams(
            collective_id=0,
            vmem_limit_bytes=100 * 1024 * 1024,
        ),
    )(x, w_mat)


# device time: 163842 ns/iter; 1.8948x vs baseline; 1.0138x over previous
import jax
import jax.numpy as jnp
from jax import lax
from jax.experimental import pallas as pl
from jax.experimental.pallas import tpu as pltpu

N_DEV = 4
SEG = 4


def kernel(x, w_mat):
    m, k_per = x.shape
    k_per2, n = w_mat.shape
    assert k_per == k_per2
    m_chunk = m // N_DEV
    m_half = m_chunk // 2
    m_seg = m_half // SEG
    n_hops = N_DEV - 1

    def body(x_ref, w_ref, out_ref, acc_r, acc_l, rcv_r, rcv_l,
             rs_ssem_r, rs_ssem_l, rs_rsem_r, rs_rsem_l,
             ag_ssem_r, ag_ssem_l, ag_rsem_r, ag_rsem_l):
        my_pos = lax.axis_index("i")
        left = (my_pos - 1) % N_DEV
        right = (my_pos + 1) % N_DEV

        def partial_rows(row0, nrows):
            xs = x_ref[pl.ds(row0, nrows), :]
            return jnp.dot(xs, w_ref[:, :], preferred_element_type=jnp.float32)

        def silu(y):
            return y * (1.0 / (1.0 + jnp.exp(-y)))

        def slab_row0(c, dirn):
            return c * m_chunk + dirn * m_half

        def rs_desc(dirn, h, s):
            acc, rcv = (acc_r, rcv_r) if dirn == 0 else (acc_l, rcv_l)
            ssem = rs_ssem_r if dirn == 0 else rs_ssem_l
            rsem = rs_rsem_r if dirn == 0 else rs_rsem_l
            tgt = right if dirn == 0 else left
            return pltpu.make_async_remote_copy(
                src_ref=acc.at[h, pl.ds(s * m_seg, m_seg), :],
                dst_ref=rcv.at[h, pl.ds(s * m_seg, m_seg), :],
                send_sem=ssem.at[h, s],
                recv_sem=rsem.at[h, s],
                device_id=(tgt,),
                device_id_type=pl.DeviceIdType.MESH,
            )

        def ag_desc(dirn, g, s):
            if dirn == 0:
                sc = (my_pos - g + 1) % N_DEV
                tgt = right
                ssem, rsem = ag_ssem_r, ag_rsem_r
            else:
                sc = (my_pos + g - 1) % N_DEV
                tgt = left
                ssem, rsem = ag_ssem_l, ag_rsem_l
            row0 = slab_row0(sc, dirn) + s * m_seg
            sl = out_ref.at[pl.ds(row0, m_seg), :]
            return pltpu.make_async_remote_copy(
                src_ref=sl, dst_ref=sl,
                send_sem=ssem.at[g, s],
                recv_sem=rsem.at[g, s],
                device_id=(tgt,),
                device_id_type=pl.DeviceIdType.MESH,
            )

        acc_r[0, :, :] = partial_rows(slab_row0(my_pos, 0), m_half)
        acc_l[0, :, :] = partial_rows(slab_row0(my_pos, 1), m_half)

        barrier_sem = pltpu.get_barrier_semaphore()
        pl.semaphore_signal(barrier_sem, inc=1, device_id=(left,),
                            device_id_type=pl.DeviceIdType.MESH)
        pl.semaphore_signal(barrier_sem, inc=1, device_id=(right,),
                            device_id_type=pl.DeviceIdType.MESH)
        pl.semaphore_wait(barrier_sem, 2)

        for s in range(SEG):
            rs_desc(0, 0, s).start()
            rs_desc(1, 0, s).start()
        for h in range(n_hops):
            cr = (my_pos - h - 1) % N_DEV
            cl = (my_pos + h + 1) % N_DEV
            for s in range(SEG):
                for dirn, c in ((0, cr), (1, cl)):
                    row0 = slab_row0(c, dirn) + s * m_seg
                    p = partial_rows(row0, m_seg)
                    rs_desc(dirn, h, s).wait_recv()
                    rcv = rcv_r if dirn == 0 else rcv_l
                    val = p + rcv[h, pl.ds(s * m_seg, m_seg), :]
                    if h < n_hops - 1:
                        acc = acc_r if dirn == 0 else acc_l
                        acc[h + 1, pl.ds(s * m_seg, m_seg), :] = val
                        rs_desc(dirn, h + 1, s).start()
                    else:
                        out_ref[pl.ds(row0, m_seg), :] = silu(val)
                        ag_desc(dirn, 0, s).start()

        for g in range(n_hops):
            for s in range(SEG):
                for dirn in (0, 1):
                    ag_desc(dirn, g, s).wait_recv()
                    if g < n_hops - 1:
                        ag_desc(dirn, g + 1, s).start()

        for h in range(n_hops):
            for s in range(SEG):
                for dirn in (0, 1):
                    rs_desc(dirn, h, s).wait_send()
                    ag_desc(dirn, h, s).wait_send()

    return pl.pallas_call(
        body,
        out_shape=jax.ShapeDtypeStruct((m, n), jnp.float32),
        in_specs=[
            pl.BlockSpec(memory_space=pltpu.VMEM),
            pl.BlockSpec(memory_space=pltpu.VMEM),
        ],
        out_specs=pl.BlockSpec(memory_space=pltpu.VMEM),
        scratch_shapes=[
            pltpu.VMEM((n_hops, m_half, n), jnp.float32),
            pltpu.VMEM((n_hops, m_half, n), jnp.float32),
            pltpu.VMEM((n_hops, m_half, n), jnp.float32),
            pltpu.VMEM((n_hops, m_half, n), jnp.float32),
            pltpu.SemaphoreType.DMA((n_hops, SEG)),
            pltpu.SemaphoreType.DMA((n_hops, SEG)),
            pltpu.SemaphoreType.DMA((n_hops, SEG)),
            pltpu.SemaphoreType.DMA((n_hops, SEG)),
            pltpu.SemaphoreType.DMA((n_hops, SEG)),
            pltpu.SemaphoreType.DMA((n_hops, SEG)),
            pltpu.SemaphoreType.DMA((n_hops, SEG)),
            pltpu.SemaphoreType.DMA((n_hops, SEG)),
        ],
        compiler_params=pltpu.CompilerParams(
            collective_id=0,
            vmem_limit_bytes=100 * 1024 * 1024,
        ),
    )(x, w_mat)
